# Initial kernel scaffold; baseline (speedup 1.0000x reference)
import jax
import jax.numpy as jnp
from jax.experimental import pallas as pl


def kernel(lm_emb, lm_emb_valid, layer_embedding, nonsense_embedding):
    raise NotImplementedError("write your pallas kernel here")


# speedup vs baseline: 5.0267x; 5.0267x over previous
import functools

import jax
import jax.numpy as jnp
from jax import lax
from jax.experimental import pallas as pl
from jax.experimental.pallas import tpu as pltpu
from jax.experimental.pallas import tpu_sc as plsc

B, L, T, D = 8, 6, 512, 1024
LAST_LAYER = 6
MAX_ITERS = 6
N_ROWS = LAST_LAYER * MAX_ITERS
N_MEM = N_ROWS + 1
MSK_PAD = 48
LANES = 16
WPB = 4
ROWS_PER_W = N_ROWS // WPB


def _sc_rolling(lm2d, valid_flat, layer_embedding, nonsense_embedding):
    mesh = plsc.VectorSubcoreMesh(core_axis_name="c", subcore_axis_name="s")

    @functools.partial(
        pl.kernel,
        out_type=[
            jax.ShapeDtypeStruct((B * N_MEM * D,), jnp.float32),
            jax.ShapeDtypeStruct((B * MSK_PAD,), jnp.int32),
        ],
        mesh=mesh,
        compiler_params=pltpu.CompilerParams(needs_layout_passes=False),
        scratch_types=[
            pltpu.VMEM((T,), jnp.int32),
            pltpu.VMEM((L, D), jnp.float32),
            pltpu.VMEM((LANES, D), jnp.float32),
            pltpu.VMEM((LANES,), jnp.int32),
            pltpu.VMEM((MSK_PAD,), jnp.int32),
            pltpu.SemaphoreType.DMA,
        ],
    )
    def k(lm_hbm, valid_hbm, layer_hbm, non_hbm, mem_hbm, msk_hbm,
          valid_v, layer_v, rows_v, red_v, msk_v, sem):
        wid = lax.axis_index("s") * mesh.num_cores + lax.axis_index("c")
        b = wid // WPB
        j = wid % WPB

        pltpu.sync_copy(valid_hbm.at[pl.ds(b * T, T)], valid_v)
        pltpu.sync_copy(layer_hbm, layer_v)

        lane = lax.iota(jnp.int32, LANES)

        def chunk_min(i, cur):
            c = valid_v[pl.ds(i * LANES, LANES)]
            cand = jnp.where(c != 0, lane + i * LANES, T)
            return jnp.minimum(cur, cand)

        minv = lax.fori_loop(0, T // LANES, chunk_min,
                             jnp.full((LANES,), T, jnp.int32))
        for s in (8, 4, 2, 1):
            red_v[...] = minv
            minv = jnp.minimum(minv, plsc.load_gather(red_v, [lane ^ s]))
        first = jnp.where(minv >= T, 0, minv)

        r = j * ROWS_PER_W + jnp.minimum(lane, ROWS_PER_W - 1)
        o = r // LAST_LAYER
        l = r % LAST_LAYER
        t = jnp.minimum(first + o, T - 1)
        row_ids = (b * L + l) * T + t
        pltpu.async_copy(lm_hbm.at[row_ids], rows_v, sem).wait()

        for kk in range(ROWS_PER_W):
            rk = j * ROWS_PER_W + kk
            lk = rk % LAST_LAYER

            def add_chunk(i, _, kk=kk, lk=lk):
                sl = pl.ds(i * LANES, LANES)
                rows_v[kk, sl] = rows_v[kk, sl] + layer_v[lk, sl]
                return 0

            lax.fori_loop(0, D // LANES, add_chunk, 0)
            pltpu.sync_copy(rows_v.at[kk],
                            mem_hbm.at[pl.ds((b * N_MEM + rk) * D, D)])

        @pl.when(j == 0)
        def _():
            pltpu.sync_copy(non_hbm, rows_v.at[pl.ds(LANES - 1, 1)])
            pltpu.sync_copy(rows_v.at[LANES - 1],
                            mem_hbm.at[pl.ds((b * N_MEM + N_MEM - 1) * D, D)])
            for i in range(MSK_PAD // LANES):
                p = lane + i * LANES
                po = p // LAST_LAYER
                pt = jnp.minimum(first + po, T - 1)
                v = plsc.load_gather(valid_v, [pt])
                msk_v[pl.ds(i * LANES, LANES)] = jnp.where(
                    (v == 0) & (p < N_ROWS), 1, 0)
            pltpu.sync_copy(msk_v, msk_hbm.at[pl.ds(b * MSK_PAD, MSK_PAD)])

    return k(lm2d, valid_flat, layer_embedding, nonsense_embedding)


def kernel(lm_emb, lm_emb_valid, layer_embedding, nonsense_embedding):
    mem_flat, msk_i32 = _sc_rolling(
        lm_emb.reshape(B * L * T, D),
        lm_emb_valid.astype(jnp.int32).reshape(-1),
        layer_embedding,
        nonsense_embedding)
    mem = mem_flat.reshape(B, N_MEM, D)
    msk = msk_i32.reshape(B, MSK_PAD)[:, :N_MEM] != 0
    return mem, msk


# speedup vs baseline: 5.4663x; 1.0875x over previous
import functools

import jax
import jax.numpy as jnp
from jax import lax
from jax.experimental import pallas as pl
from jax.experimental.pallas import tpu as pltpu
from jax.experimental.pallas import tpu_sc as plsc

B, L, T, D = 8, 6, 512, 1024
LAST_LAYER = 6
MAX_ITERS = 6
N_ROWS = LAST_LAYER * MAX_ITERS
N_MEM = N_ROWS + 1
MSK_PAD = 48
LANES = 16
WPB = 4
ROWS_PER_W = N_ROWS // WPB


def _sc_rolling(lm2d, valid_flat, layer_embedding, nonsense_embedding):
    mesh = plsc.VectorSubcoreMesh(core_axis_name="c", subcore_axis_name="s")

    @functools.partial(
        pl.kernel,
        out_type=[
            jax.ShapeDtypeStruct((B * N_MEM * D,), jnp.float32),
            jax.ShapeDtypeStruct((B * MSK_PAD,), jnp.int32),
        ],
        mesh=mesh,
        compiler_params=pltpu.CompilerParams(needs_layout_passes=False),
        scratch_types=[
            pltpu.VMEM((T,), jnp.int32),
            pltpu.VMEM((L, D), jnp.float32),
            pltpu.VMEM((LANES, D), jnp.float32),
            pltpu.VMEM((D,), jnp.float32),
            pltpu.VMEM((MSK_PAD,), jnp.int32),
            pltpu.SemaphoreType.DMA,
            pltpu.SemaphoreType.DMA,
            pltpu.SemaphoreType.DMA,
            pltpu.SemaphoreType.DMA,
        ],
    )
    def k(lm_hbm, valid_hbm, layer_hbm, non_hbm, mem_hbm, msk_hbm,
          valid_v, layer_v, rows_v, non_v, msk_v,
          sem_l, sem_v, sem_g, sem_s):
        wid = lax.axis_index("s") * mesh.num_cores + lax.axis_index("c")
        b = wid // WPB
        j = wid % WPB

        layer_cp = pltpu.async_copy(layer_hbm, layer_v, sem_l)
        valid_cp = pltpu.async_copy(valid_hbm.at[pl.ds(b * T, T)], valid_v,
                                    sem_v)

        lane = lax.iota(jnp.int32, LANES)
        valid_cp.wait()

        def chunk_min(i, cur):
            c = valid_v[pl.ds(i * LANES, LANES)]
            return jnp.minimum(cur, jnp.where(c != 0, lane + i * LANES, T))

        minv = lax.fori_loop(0, T // LANES, chunk_min,
                             jnp.full((LANES,), T, jnp.int32), unroll=4)
        first = jnp.min(minv)
        first = jnp.where(first >= T, 0, first)

        r = j * ROWS_PER_W + jnp.minimum(lane, ROWS_PER_W - 1)
        o = r // LAST_LAYER
        l = r % LAST_LAYER
        t = jnp.minimum(first + o, T - 1)
        row_ids = (b * L + l) * T + t
        gather_cp = pltpu.async_copy(lm_hbm.at[row_ids], rows_v, sem_g)

        @pl.when(j == 0)
        def _():
            pltpu.sync_copy(non_hbm, non_v)
            pltpu.sync_copy(non_v,
                            mem_hbm.at[pl.ds((b * N_MEM + N_MEM - 1) * D, D)])
            for i in range(MSK_PAD // LANES):
                p = lane + i * LANES
                pt = jnp.minimum(first + p // LAST_LAYER, T - 1)
                v = plsc.load_gather(valid_v, [pt])
                msk_v[pl.ds(i * LANES, LANES)] = jnp.where(
                    (v == 0) & (p < N_ROWS), 1, 0)
            pltpu.sync_copy(msk_v, msk_hbm.at[pl.ds(b * MSK_PAD, MSK_PAD)])

        layer_cp.wait()
        gather_cp.wait()

        stores = []
        for kk in range(ROWS_PER_W):
            rk = j * ROWS_PER_W + kk
            lk = rk % LAST_LAYER

            def add_chunk(i, _, kk=kk, lk=lk):
                sl = pl.ds(i * LANES, LANES)
                rows_v[kk, sl] = rows_v[kk, sl] + layer_v[lk, sl]
                return 0

            lax.fori_loop(0, D // LANES, add_chunk, 0, unroll=8)
            stores.append(pltpu.async_copy(
                rows_v.at[kk], mem_hbm.at[pl.ds((b * N_MEM + rk) * D, D)],
                sem_s))
        for cp in stores:
            cp.wait()

    return k(lm2d, valid_flat, layer_embedding, nonsense_embedding)


def kernel(lm_emb, lm_emb_valid, layer_embedding, nonsense_embedding):
    mem_flat, msk_i32 = _sc_rolling(
        lm_emb.reshape(B * L * T, D),
        lm_emb_valid.astype(jnp.int32).reshape(-1),
        layer_embedding,
        nonsense_embedding.reshape(-1))
    mem = mem_flat.reshape(B, N_MEM, D)
    msk = msk_i32.reshape(B, MSK_PAD)[:, :N_MEM] != 0
    return mem, msk


# speedup vs baseline: 5.6580x; 1.0351x over previous
import functools

import jax
import jax.numpy as jnp
from jax import lax
from jax.experimental import pallas as pl
from jax.experimental.pallas import tpu as pltpu
from jax.experimental.pallas import tpu_sc as plsc

B, L, T, D = 8, 6, 512, 1024
LAST_LAYER = 6
MAX_ITERS = 6
N_ROWS = LAST_LAYER * MAX_ITERS
N_MEM = N_ROWS + 1
MSK_PAD = 48
LANES = 16
WPB = 4
ROWS_PER_W = N_ROWS // WPB


def _sc_rolling(lm2d, valid_flat, layer_embedding, non_flat):
    mesh = plsc.VectorSubcoreMesh(core_axis_name="c", subcore_axis_name="s")

    @functools.partial(
        pl.kernel,
        out_type=[
            jax.ShapeDtypeStruct((B, N_MEM, D), jnp.float32),
            jax.ShapeDtypeStruct((B, MSK_PAD), jnp.int32),
        ],
        mesh=mesh,
        compiler_params=pltpu.CompilerParams(needs_layout_passes=False),
        scratch_types=[
            pltpu.VMEM((T,), jnp.int32),
            pltpu.VMEM((L, D), jnp.float32),
            pltpu.VMEM((8, D), jnp.float32),
            pltpu.VMEM((1, D), jnp.float32),
            pltpu.VMEM((LANES,), jnp.int32),
            pltpu.VMEM((1, D), jnp.float32),
            pltpu.VMEM((1, MSK_PAD), jnp.int32),
            pltpu.SemaphoreType.DMA,
            pltpu.SemaphoreType.DMA,
            pltpu.SemaphoreType.DMA,
            pltpu.SemaphoreType.DMA,
            pltpu.SemaphoreType.DMA,
        ],
    )
    def k(lm_hbm, valid_hbm, layer_hbm, non_hbm, mem_hbm, msk_hbm,
          valid_v, layer_v, rows8_v, rows1_v, idx_v, non_v, msk_v,
          sem_l, sem_v, sem_g, sem_h, sem_s):
        wid = lax.axis_index("s") * mesh.num_cores + lax.axis_index("c")
        b = wid // WPB
        j = wid % WPB

        layer_cp = pltpu.async_copy(layer_hbm, layer_v, sem_l)
        valid_cp = pltpu.async_copy(valid_hbm.at[pl.ds(b * T, T)], valid_v,
                                    sem_v)

        lane = lax.iota(jnp.int32, LANES)
        valid_cp.wait()

        def chunk_min(i, cur):
            c = valid_v[pl.ds(i * LANES, LANES)]
            return jnp.minimum(cur, jnp.where(c != 0, lane + i * LANES, T))

        minv = lax.fori_loop(0, T // LANES, chunk_min,
                             jnp.full((LANES,), T, jnp.int32), unroll=4)
        first = jnp.min(minv)
        first = jnp.where(first >= T, 0, first)

        r = j * ROWS_PER_W + jnp.minimum(lane, ROWS_PER_W - 1)
        o = r // LAST_LAYER
        l = r % LAST_LAYER
        t = jnp.minimum(first + o, T - 1)
        idx_v[...] = (b * L + l) * T + t
        g8_cp = pltpu.async_copy(lm_hbm.at[idx_v.at[pl.ds(0, 8)]],
                                 rows8_v, sem_g)
        g1_cp = pltpu.async_copy(lm_hbm.at[idx_v.at[pl.ds(8, 1)]],
                                 rows1_v, sem_h)

        @pl.when(j == 0)
        def _():
            pltpu.sync_copy(non_hbm, non_v)
            pltpu.sync_copy(non_v, mem_hbm.at[b, pl.ds(N_MEM - 1, 1)])
            for i in range(MSK_PAD // LANES):
                p = lane + i * LANES
                pt = jnp.minimum(first + p // LAST_LAYER, T - 1)
                v = plsc.load_gather(valid_v, [pt])
                msk_v[0, pl.ds(i * LANES, LANES)] = jnp.where(
                    (v == 0) & (p < N_ROWS), 1, 0)
            pltpu.sync_copy(msk_v, msk_hbm.at[pl.ds(b, 1)])

        layer_cp.wait()
        g8_cp.wait()

        def add_row(buf, kk, lk):
            def add_chunk(i, _):
                sl = pl.ds(i * LANES, LANES)
                buf[kk, sl] = buf[kk, sl] + layer_v[lk, sl]
                return 0
            lax.fori_loop(0, D // LANES, add_chunk, 0, unroll=8)

        stores = []
        for kk in range(ROWS_PER_W):
            rk = j * ROWS_PER_W + kk
            lk = rk % LAST_LAYER
            buf, bk = (rows8_v, kk) if kk < 8 else (rows1_v, 0)
            if kk == ROWS_PER_W - 1:
                g1_cp.wait()
            add_row(buf, bk, lk)
            stores.append(pltpu.async_copy(
                buf.at[pl.ds(bk, 1)], mem_hbm.at[b, pl.ds(rk, 1)], sem_s))
        for cp in stores:
            cp.wait()

    return k(lm2d, valid_flat, layer_embedding, non_flat)


def kernel(lm_emb, lm_emb_valid, layer_embedding, nonsense_embedding):
    mem, msk_i32 = _sc_rolling(
        lm_emb.reshape(B * L * T, D),
        lm_emb_valid.astype(jnp.int32).reshape(-1),
        layer_embedding,
        nonsense_embedding)
    msk = msk_i32[:, :N_MEM] != 0
    return mem, msk


# speedup vs baseline: 6.1234x; 1.0823x over previous
import functools

import jax
import jax.numpy as jnp
from jax import lax
from jax.experimental import pallas as pl
from jax.experimental.pallas import tpu as pltpu
from jax.experimental.pallas import tpu_sc as plsc

B, L, T, D = 8, 6, 512, 1024
LAST_LAYER = 6
MAX_ITERS = 6
N_ROWS = LAST_LAYER * MAX_ITERS
N_MEM = N_ROWS + 1
MSK_PAD = 48
LANES = 16
WPB = 4
ROWS_PER_W = N_ROWS // WPB


def _sc_rolling(lm2d, valid2d, layer_embedding, non2d):
    mesh = plsc.VectorSubcoreMesh(core_axis_name="c", subcore_axis_name="s")

    @functools.partial(
        pl.kernel,
        out_type=[
            jax.ShapeDtypeStruct((N_MEM, B, D), jnp.float32),
            jax.ShapeDtypeStruct((B, MSK_PAD), jnp.int32),
        ],
        mesh=mesh,
        compiler_params=pltpu.CompilerParams(needs_layout_passes=False),
        scratch_types=[
            pltpu.VMEM((1, T), jnp.int32),
            pltpu.VMEM((L, D), jnp.float32),
            pltpu.VMEM((8, D), jnp.float32),
            pltpu.VMEM((1, D), jnp.float32),
            pltpu.VMEM((LANES,), jnp.int32),
            pltpu.VMEM((1, D), jnp.float32),
            pltpu.VMEM((1, MSK_PAD), jnp.int32),
            pltpu.SemaphoreType.DMA,
            pltpu.SemaphoreType.DMA,
            pltpu.SemaphoreType.DMA,
            pltpu.SemaphoreType.DMA,
            pltpu.SemaphoreType.DMA,
        ],
    )
    def k(lm_hbm, valid_hbm, layer_hbm, non_hbm, mem_hbm, msk_hbm,
          valid_v, layer_v, rows8_v, rows1_v, gidx_v, non_v, msk_v,
          sem_l, sem_v, sem_g, sem_h, sem_s):
        wid = lax.axis_index("s") * mesh.num_cores + lax.axis_index("c")
        b = wid // WPB
        j = wid % WPB

        layer_cp = pltpu.async_copy(layer_hbm, layer_v, sem_l)
        valid_cp = pltpu.async_copy(valid_hbm.at[pl.ds(b, 1)], valid_v, sem_v)

        lane = lax.iota(jnp.int32, LANES)
        valid_cp.wait()

        def chunk_min(i, cur):
            c = valid_v[0, pl.ds(i * LANES, LANES)]
            return jnp.minimum(cur, jnp.where(c != 0, lane + i * LANES, T))

        minv = lax.fori_loop(0, T // LANES, chunk_min,
                             jnp.full((LANES,), T, jnp.int32), unroll=4)
        first = jnp.min(minv)
        first = jnp.where(first >= T, 0, first)

        r = j * ROWS_PER_W + jnp.minimum(lane, ROWS_PER_W - 1)
        l = r % LAST_LAYER
        t = jnp.minimum(first + r // LAST_LAYER, T - 1)
        gidx_v[...] = (b * L + l) * T + t
        g8_cp = pltpu.async_copy(lm_hbm.at[gidx_v.at[pl.ds(0, 8)]],
                                 rows8_v, sem_g)
        g1_cp = pltpu.async_copy(lm_hbm.at[gidx_v.at[pl.ds(8, 1)]],
                                 rows1_v, sem_h)

        @pl.when(j == 0)
        def _():
            pltpu.sync_copy(non_hbm, non_v)
            pltpu.sync_copy(non_v, mem_hbm.at[N_MEM - 1, pl.ds(b, 1)])
            for i in range(MSK_PAD // LANES):
                p = lane + i * LANES
                pt = jnp.minimum(first + p // LAST_LAYER, T - 1)
                v = plsc.load_gather(valid_v, [jnp.zeros((LANES,), jnp.int32),
                                               pt])
                msk_v[0, pl.ds(i * LANES, LANES)] = jnp.where(
                    (v == 0) & (p < N_ROWS), 1, 0)
            pltpu.sync_copy(msk_v, msk_hbm.at[pl.ds(b, 1)])

        layer_cp.wait()
        g8_cp.wait()

        def add_row(buf, kk, lk):
            def add_chunk(i, _):
                sl = pl.ds(i * LANES, LANES)
                buf[kk, sl] = buf[kk, sl] + layer_v[lk, sl]
                return 0
            lax.fori_loop(0, D // LANES, add_chunk, 0, unroll=8)

        stores = []
        for kk in range(ROWS_PER_W):
            rk = j * ROWS_PER_W + kk
            lk = rk % LAST_LAYER
            buf, bk = (rows8_v, kk) if kk < 8 else (rows1_v, 0)
            if kk == ROWS_PER_W - 1:
                g1_cp.wait()
            add_row(buf, bk, lk)
            stores.append(pltpu.async_copy(
                buf.at[pl.ds(bk, 1)], mem_hbm.at[rk, pl.ds(b, 1)], sem_s))
        for cp in stores:
            cp.wait()

    return k(lm2d, valid2d, layer_embedding, non2d)


def kernel(lm_emb, lm_emb_valid, layer_embedding, nonsense_embedding):
    mem_t, msk_i32 = _sc_rolling(
        lm_emb.reshape(B * L * T, D),
        lm_emb_valid.astype(jnp.int32),
        layer_embedding,
        nonsense_embedding)
    mem = jnp.transpose(mem_t, (1, 0, 2))
    msk = msk_i32[:, :N_MEM] != 0
    return mem, msk
